# baseline (speedup 1.0000x reference)
import jax
import jax.numpy as jnp
from jax.experimental import pallas as pl

_N = 100000
_D = 128
_BLOCK = 2000


def _sum_body(x_ref, o_ref):
    @pl.when(pl.program_id(0) == 0)
    def _():
        o_ref[...] = jnp.zeros_like(o_ref)

    o_ref[...] += jnp.sum(x_ref[...], axis=0, keepdims=True)


def kernel(x):
    grid = _N // _BLOCK
    out = pl.pallas_call(
        _sum_body,
        grid=(grid,),
        in_specs=[pl.BlockSpec((_BLOCK, _D), lambda i: (i, 0))],
        out_specs=pl.BlockSpec((1, _D), lambda i: (0, 0)),
        out_shape=jax.ShapeDtypeStruct((1, _D), jnp.float32),
    )(x)
    return out


# speedup vs baseline: 1.5467x; 1.5467x over previous
import jax
import jax.numpy as jnp
from jax.experimental import pallas as pl
from jax.experimental.pallas import tpu as pltpu

_N = 100000
_D = 128
_BLOCK = 5000


def _sum_body(x_ref, o_ref, acc_ref):
    @pl.when(pl.program_id(0) == 0)
    def _():
        acc_ref[...] = jnp.zeros_like(acc_ref)

    acc_ref[...] += jnp.sum(x_ref[...].reshape(-1, 8, _D), axis=0)

    @pl.when(pl.program_id(0) == pl.num_programs(0) - 1)
    def _():
        o_ref[...] = jnp.sum(acc_ref[...], axis=0, keepdims=True)


def kernel(x):
    grid = _N // _BLOCK
    out = pl.pallas_call(
        _sum_body,
        grid=(grid,),
        in_specs=[pl.BlockSpec((_BLOCK, _D), lambda i: (i, 0))],
        out_specs=pl.BlockSpec((1, _D), lambda i: (0, 0)),
        out_shape=jax.ShapeDtypeStruct((1, _D), jnp.float32),
        scratch_shapes=[pltpu.VMEM((8, _D), jnp.float32)],
    )(x)
    return out


# speedup vs baseline: 1.8286x; 1.1822x over previous
import jax
import jax.numpy as jnp
from jax.experimental import pallas as pl
from jax.experimental.pallas import tpu as pltpu

_N = 100000
_D = 128
_BLOCK = 5000


def _sum_body(x_ref, o_ref, acc_ref):
    @pl.when(pl.program_id(0) == 0)
    def _():
        acc_ref[...] = jnp.zeros_like(acc_ref)

    acc_ref[...] += jnp.sum(x_ref[...].reshape(-1, 40, _D), axis=0)

    @pl.when(pl.program_id(0) == pl.num_programs(0) - 1)
    def _():
        o_ref[...] = jnp.sum(acc_ref[...], axis=0, keepdims=True)


def kernel(x):
    grid = _N // _BLOCK
    out = pl.pallas_call(
        _sum_body,
        grid=(grid,),
        in_specs=[pl.BlockSpec((_BLOCK, _D), lambda i: (i, 0))],
        out_specs=pl.BlockSpec((1, _D), lambda i: (0, 0)),
        out_shape=jax.ShapeDtypeStruct((1, _D), jnp.float32),
        scratch_shapes=[pltpu.VMEM((40, _D), jnp.float32)],
    )(x)
    return out
